# baseline (speedup 1.0000x reference)
import jax
import jax.numpy as jnp
from jax import lax
from jax.experimental import pallas as pl
from jax.experimental.pallas import tpu as pltpu
from jax.experimental.pallas import tpu_sc as plsc

N = 10000
HID = 128
H = 8
D = 16
T = 2
E = 160000

NC = 2
NS = 16
L = 16
NW = NC * NS
C = 32
NCHUNK = E // C
STRIPE = 640
STRIPE_LAST = N - (NS - 1) * STRIPE
W2P = 2 * HID
OW = HID + H


_R = 2000


def _tables_body(x_ref, wq_ref, wk_ref, wmsg_ref, bmsg_ref,
                 s0_ref, s1_ref, t0_ref, t1_ref):
    scale = D ** -0.5
    xb = x_ref[...]
    outs = (s0_ref, s1_ref, t0_ref, t1_ref)
    for t in range(T):
        w_src = jnp.concatenate([wk_ref[t], wmsg_ref[t, :HID]], axis=1)
        outs[t][...] = jnp.dot(xb, w_src, preferred_element_type=jnp.float32)
        w_tgt = jnp.concatenate(
            [wq_ref[t] * scale, wmsg_ref[t, HID:]], axis=1)
        bias = jnp.concatenate(
            [jnp.zeros((HID,), jnp.float32), bmsg_ref[t]])
        outs[T + t][...] = (
            jnp.dot(xb, w_tgt, preferred_element_type=jnp.float32)
            + bias[None, :])


def _build_tables(x, Wq, Wk, Wmsg, bmsg):
    tab = jax.ShapeDtypeStruct((N, W2P), jnp.float32)
    return pl.pallas_call(
        _tables_body,
        grid=(N // _R,),
        in_specs=[
            pl.BlockSpec((_R, HID), lambda i: (i, 0)),
            pl.BlockSpec((T, HID, HID), lambda i: (0, 0, 0)),
            pl.BlockSpec((T, HID, HID), lambda i: (0, 0, 0)),
            pl.BlockSpec((T, 2 * HID, HID), lambda i: (0, 0, 0)),
            pl.BlockSpec((T, HID), lambda i: (0, 0)),
        ],
        out_specs=[pl.BlockSpec((_R, W2P), lambda i: (i, 0))] * 4,
        out_shape=[tab, tab, tab, tab],
    )(x, Wq, Wk, Wmsg, bmsg)


def _edge_body(s0, s1, t0, t1, srcs, tgts, znum,
               num_out,
               sidx0, tidx0, srow0, trow0, out0, ex0,
               sidx1, tidx1, srow1, trow1, out1, ex1,
               num_acc, gsem0, gsem1, ssem0, ssem1):
    c = lax.axis_index("c")
    s = lax.axis_index("s")
    w = s * NC + c
    bufs = ((sidx0, tidx0, srow0, trow0, out0, ex0, gsem0, ssem0),
            (sidx1, tidx1, srow1, trow1, out1, ex1, gsem1, ssem1))

    off = pl.multiple_of(s * STRIPE, 8)

    @pl.when(s < NS - 1)
    def _():
        pltpu.sync_copy(znum.at[pl.ds(off, STRIPE)],
                        num_acc.at[pl.ds(off, STRIPE)])

    @pl.when(s == NS - 1)
    def _():
        base = (NS - 1) * STRIPE
        pltpu.sync_copy(znum.at[pl.ds(base, STRIPE_LAST)],
                        num_acc.at[pl.ds(base, STRIPE_LAST)])

    plsc.subcore_barrier()

    lanes = lax.iota(jnp.int32, L)
    lanes_mod_h = lanes % H
    lanes_div_h = lanes // H

    def load_idx(t, g, bs):
        base = pl.multiple_of(g * C, C)
        pltpu.sync_copy(srcs.at[t, pl.ds(base, C)], bs[0])
        pltpu.sync_copy(tgts.at[t, pl.ds(base, C)], bs[1])

    def issue_gathers(stab, ttab, bs):
        pltpu.make_async_copy(stab.at[bs[0]], bs[2], bs[6]).start()
        pltpu.make_async_copy(ttab.at[bs[1]], bs[3], bs[6]).start()

    def wait_gathers(stab, ttab, bs):
        pltpu.make_async_copy(stab.at[bs[0]], bs[2], bs[6]).wait()
        pltpu.make_async_copy(ttab.at[bs[1]], bs[3], bs[6]).wait()

    def issue_scatters(bs):
        pltpu.async_copy(bs[4], num_acc.at[bs[1]], bs[7], add=True)

    def wait_scatters(bs):
        pltpu.make_async_copy(bs[4], num_acc.at[bs[1]], bs[7]).wait()

    def compute(bs):
        _, _, srow_v, trow_v, out_v, ex_v = bs[:6]

        def group_body(j, _):
            evec = j * L + lanes
            for h in range(H):
                acc0 = jnp.zeros((L,), jnp.float32)
                acc1 = jnp.zeros((L,), jnp.float32)
                for d in range(0, L, 2):
                    c0 = h * L + ((lanes + d) & (L - 1))
                    c1 = h * L + ((lanes + d + 1) & (L - 1))
                    acc0 = acc0 + (plsc.load_gather(trow_v, [evec, c0])
                                   * plsc.load_gather(srow_v, [evec, c0]))
                    acc1 = acc1 + (plsc.load_gather(trow_v, [evec, c1])
                                   * plsc.load_gather(srow_v, [evec, c1]))
                exv = jnp.exp(acc0 + acc1)
                plsc.store_scatter(
                    ex_v, [evec, jnp.full((L,), h, jnp.int32)], exv)
            return 0

        lax.fori_loop(0, C // L, group_body, 0, unroll=False)

        def pair_body(p, _):
            e0 = 2 * p
            rows = e0 + lanes_div_h
            exrow = plsc.load_gather(ex_v, [rows, lanes_mod_h])
            plsc.store_scatter(out_v, [rows, HID + lanes_mod_h], exrow)
            for k in range(2):
                e = e0 + k
                for h in range(H):
                    av = srow_v[e, pl.ds(HID + h * L, L)]
                    bv = trow_v[e, pl.ds(HID + h * L, L)]
                    m = jnp.maximum(av + bv, 0.0)
                    out_v[e, pl.ds(h * L, L)] = m * exrow[k * H + h]
            return 0

        lax.fori_loop(0, C // 2, pair_body, 0, unroll=2)

    def process_type(src_tab, tgt_tab, t):
        n_i = (NCHUNK - w + NW - 1) // NW
        load_idx(t, w, bufs[0])
        issue_gathers(src_tab, tgt_tab, bufs[0])

        def chunk_body(i, _):
            def stage(cur, nxt):
                @pl.when(i + 1 < n_i)
                def _():
                    @pl.when(i > 0)
                    def _():
                        wait_scatters(nxt)

                    load_idx(t, w + (i + 1) * NW, nxt)
                    issue_gathers(src_tab, tgt_tab, nxt)

                wait_gathers(src_tab, tgt_tab, cur)
                compute(cur)
                issue_scatters(cur)

            @pl.when(i % 2 == 0)
            def _():
                stage(bufs[0], bufs[1])

            @pl.when(i % 2 == 1)
            def _():
                stage(bufs[1], bufs[0])

            return 0

        lax.fori_loop(0, n_i, chunk_body, 0, unroll=False)
        wait_scatters(bufs[0])
        wait_scatters(bufs[1])

    process_type(s0, t0, 0)
    process_type(s1, t1, 1)

    plsc.subcore_barrier()
    @pl.when(s < NS - 1)
    def _():
        pltpu.sync_copy(num_acc.at[pl.ds(off, STRIPE)],
                        num_out.at[c, pl.ds(off, STRIPE)])

    @pl.when(s == NS - 1)
    def _():
        base = (NS - 1) * STRIPE
        pltpu.sync_copy(num_acc.at[pl.ds(base, STRIPE_LAST)],
                        num_out.at[c, pl.ds(base, STRIPE_LAST)])


def _edge_pass(s0, s1, t0, t1, srcs, tgts, znum):
    mesh = plsc.VectorSubcoreMesh(core_axis_name="c", subcore_axis_name="s")
    f = pl.kernel(
        _edge_body,
        out_type=jax.ShapeDtypeStruct((NC, N, OW), jnp.float32),
        mesh=mesh,
        scratch_types=(
            [
                pltpu.VMEM((C,), jnp.int32),
                pltpu.VMEM((C,), jnp.int32),
                pltpu.VMEM((C, W2P), jnp.float32),
                pltpu.VMEM((C, W2P), jnp.float32),
                pltpu.VMEM((C, OW), jnp.float32),
                pltpu.VMEM((C, H + 1), jnp.float32),
            ] * 2
            + [
                pltpu.VMEM_SHARED((N, OW), jnp.float32),
                pltpu.SemaphoreType.DMA,
                pltpu.SemaphoreType.DMA,
                pltpu.SemaphoreType.DMA,
                pltpu.SemaphoreType.DMA,
            ]
        ),
        compiler_params=pltpu.CompilerParams(use_tc_tiling_on_sc=False,
                                             needs_layout_passes=False),
    )
    return f(s0, s1, t0, t1, srcs, tgts, znum)


def _norm_body(acc_ref, out_ref):
    acc = acc_ref[0] + acc_ref[1]
    num = acc[:, :HID]
    den = acc[:, HID:]
    recip = jnp.where(den > 0, 1.0 / den, 0.0)
    row = lax.broadcasted_iota(jnp.int32, (H, HID), 0)
    col = lax.broadcasted_iota(jnp.int32, (H, HID), 1)
    emat = (col // L == row).astype(jnp.float32)
    out_ref[...] = num * jnp.dot(recip, emat,
                                 preferred_element_type=jnp.float32)


def _normalize(num_part):
    return pl.pallas_call(
        _norm_body,
        grid=(N // _R,),
        in_specs=[pl.BlockSpec((NC, _R, OW), lambda i: (0, i, 0))],
        out_specs=pl.BlockSpec((_R, HID), lambda i: (i, 0)),
        out_shape=jax.ShapeDtypeStruct((N, HID), jnp.float32),
    )(num_part)


def kernel(x, adj_lists, Wq, Wk, Wmsg, bmsg):
    srcs = adj_lists[..., 0]
    tgts = adj_lists[..., 1]
    s0, s1, t0, t1 = _build_tables(x, Wq, Wk, Wmsg, bmsg)
    znum = jnp.zeros((N, OW), jnp.float32)
    num_part = _edge_pass(s0, s1, t0, t1, srcs, tgts, znum)
    return _normalize(num_part)


# speedup vs baseline: 1.2497x; 1.2497x over previous
import jax
import jax.numpy as jnp
from jax import lax
from jax.experimental import pallas as pl
from jax.experimental.pallas import tpu as pltpu
from jax.experimental.pallas import tpu_sc as plsc

N = 10000
HID = 128
H = 8
D = 16
T = 2
E = 160000

NC = 2
NS = 16
L = 16
NW = NC * NS
C = 32
NCHUNK = E // C
STRIPE = 640
STRIPE_LAST = N - (NS - 1) * STRIPE
W2P = 2 * HID
OW = HID + H


_R = 2000


def _tables_body(x_ref, wq_ref, wk_ref, wmsg_ref, bmsg_ref,
                 s0_ref, s1_ref, t0_ref, t1_ref):
    scale = D ** -0.5
    xb = x_ref[...]
    outs = (s0_ref, s1_ref, t0_ref, t1_ref)
    for t in range(T):
        w_src = jnp.concatenate([wk_ref[t], wmsg_ref[t, :HID]], axis=1)
        outs[t][...] = jnp.dot(xb, w_src, preferred_element_type=jnp.float32)
        w_tgt = jnp.concatenate(
            [wq_ref[t] * scale, wmsg_ref[t, HID:]], axis=1)
        bias = jnp.concatenate(
            [jnp.zeros((HID,), jnp.float32), bmsg_ref[t]])
        outs[T + t][...] = (
            jnp.dot(xb, w_tgt, preferred_element_type=jnp.float32)
            + bias[None, :])


def _build_tables(x, Wq, Wk, Wmsg, bmsg):
    tab = jax.ShapeDtypeStruct((N, W2P), jnp.float32)
    return pl.pallas_call(
        _tables_body,
        grid=(N // _R,),
        in_specs=[
            pl.BlockSpec((_R, HID), lambda i: (i, 0)),
            pl.BlockSpec((T, HID, HID), lambda i: (0, 0, 0)),
            pl.BlockSpec((T, HID, HID), lambda i: (0, 0, 0)),
            pl.BlockSpec((T, 2 * HID, HID), lambda i: (0, 0, 0)),
            pl.BlockSpec((T, HID), lambda i: (0, 0)),
        ],
        out_specs=[pl.BlockSpec((_R, W2P), lambda i: (i, 0))] * 4,
        out_shape=[tab, tab, tab, tab],
    )(x, Wq, Wk, Wmsg, bmsg)


def _edge_body(s0, s1, t0, t1, srcs, tgts, znum,
               num_out,
               sidx0, tidx0, srow0, trow0, out0, ex0, tcp0,
               sidx1, tidx1, srow1, trow1, out1, ex1, tcp1,
               num_acc, gsem0, gsem1, ssem0, ssem1, isem0, isem1):
    c = lax.axis_index("c")
    s = lax.axis_index("s")
    w = s * NC + c
    bufs = ((sidx0, tidx0, srow0, trow0, out0, ex0, gsem0, ssem0,
             tcp0, isem0),
            (sidx1, tidx1, srow1, trow1, out1, ex1, gsem1, ssem1,
             tcp1, isem1))

    off = pl.multiple_of(s * STRIPE, 8)

    @pl.when(s < NS - 1)
    def _():
        pltpu.sync_copy(znum.at[pl.ds(off, STRIPE)],
                        num_acc.at[pl.ds(off, STRIPE)])

    @pl.when(s == NS - 1)
    def _():
        base = (NS - 1) * STRIPE
        pltpu.sync_copy(znum.at[pl.ds(base, STRIPE_LAST)],
                        num_acc.at[pl.ds(base, STRIPE_LAST)])

    plsc.subcore_barrier()

    lanes = lax.iota(jnp.int32, L)
    lanes_mod_h = lanes % H
    lanes_div_h = lanes // H

    def load_idx(t, g, bs):
        base = pl.multiple_of(g * C, C)
        pltpu.sync_copy(srcs.at[t, pl.ds(base, C)], bs[0])
        pltpu.sync_copy(tgts.at[t, pl.ds(base, C)], bs[1])

    def load_idx_async(t, g, bs):
        base = pl.multiple_of(g * C, C)
        pltpu.make_async_copy(srcs.at[t, pl.ds(base, C)], bs[0],
                              bs[9]).start()
        pltpu.make_async_copy(tgts.at[t, pl.ds(base, C)], bs[1],
                              bs[9]).start()

    def wait_idx(t, g, bs):
        base = pl.multiple_of(g * C, C)
        pltpu.make_async_copy(srcs.at[t, pl.ds(base, C)], bs[0],
                              bs[9]).wait()
        pltpu.make_async_copy(tgts.at[t, pl.ds(base, C)], bs[1],
                              bs[9]).wait()

    def issue_gathers(stab, ttab, bs):
        pltpu.make_async_copy(stab.at[bs[0]], bs[2], bs[6]).start()
        pltpu.make_async_copy(ttab.at[bs[1]], bs[3], bs[6]).start()

    def wait_gathers(stab, ttab, bs):
        pltpu.make_async_copy(stab.at[bs[0]], bs[2], bs[6]).wait()
        pltpu.make_async_copy(ttab.at[bs[1]], bs[3], bs[6]).wait()

    def snapshot_tidx(bs):
        for q in range(C // L):
            bs[8][pl.ds(q * L, L)] = bs[1][pl.ds(q * L, L)]

    def issue_scatters(bs):
        pltpu.async_copy(bs[4], num_acc.at[bs[8]], bs[7], add=True)

    def wait_scatters(bs):
        pltpu.make_async_copy(bs[4], num_acc.at[bs[8]], bs[7]).wait()

    def compute(bs):
        _, _, srow_v, trow_v, out_v, ex_v = bs[:6]

        def group_body(j, _):
            evec = j * L + lanes
            for h in range(H):
                acc0 = jnp.zeros((L,), jnp.float32)
                acc1 = jnp.zeros((L,), jnp.float32)
                for d in range(0, L, 2):
                    c0 = h * L + ((lanes + d) & (L - 1))
                    c1 = h * L + ((lanes + d + 1) & (L - 1))
                    acc0 = acc0 + (plsc.load_gather(trow_v, [evec, c0])
                                   * plsc.load_gather(srow_v, [evec, c0]))
                    acc1 = acc1 + (plsc.load_gather(trow_v, [evec, c1])
                                   * plsc.load_gather(srow_v, [evec, c1]))
                exv = jnp.exp(acc0 + acc1)
                plsc.store_scatter(
                    ex_v, [evec, jnp.full((L,), h, jnp.int32)], exv)
            return 0

        lax.fori_loop(0, C // L, group_body, 0, unroll=False)

        def pair_body(p, _):
            e0 = 2 * p
            rows = e0 + lanes_div_h
            exrow = plsc.load_gather(ex_v, [rows, lanes_mod_h])
            plsc.store_scatter(out_v, [rows, HID + lanes_mod_h], exrow)
            for k in range(2):
                e = e0 + k
                for h in range(H):
                    av = srow_v[e, pl.ds(HID + h * L, L)]
                    bv = trow_v[e, pl.ds(HID + h * L, L)]
                    m = jnp.maximum(av + bv, 0.0)
                    out_v[e, pl.ds(h * L, L)] = m * exrow[k * H + h]
            return 0

        lax.fori_loop(0, C // 2, pair_body, 0, unroll=False)

    def process_type(src_tab, tgt_tab, t):
        n_i = (NCHUNK - w + NW - 1) // NW
        load_idx(t, w, bufs[0])
        issue_gathers(src_tab, tgt_tab, bufs[0])
        load_idx_async(t, w + NW, bufs[1])

        def chunk_body(i, _):
            def stage(cur, nxt):
                @pl.when(i + 1 < n_i)
                def _():
                    @pl.when(i > 0)
                    def _():
                        wait_scatters(nxt)

                    wait_idx(t, w + (i + 1) * NW, nxt)
                    issue_gathers(src_tab, tgt_tab, nxt)

                wait_gathers(src_tab, tgt_tab, cur)
                snapshot_tidx(cur)

                @pl.when(i + 2 < n_i)
                def _():
                    load_idx_async(t, w + (i + 2) * NW, cur)

                compute(cur)
                issue_scatters(cur)

            @pl.when(i % 2 == 0)
            def _():
                stage(bufs[0], bufs[1])

            @pl.when(i % 2 == 1)
            def _():
                stage(bufs[1], bufs[0])

            return 0

        lax.fori_loop(0, n_i, chunk_body, 0, unroll=False)
        wait_scatters(bufs[0])
        wait_scatters(bufs[1])

    process_type(s0, t0, 0)
    process_type(s1, t1, 1)

    plsc.subcore_barrier()
    @pl.when(s < NS - 1)
    def _():
        pltpu.sync_copy(num_acc.at[pl.ds(off, STRIPE)],
                        num_out.at[c, pl.ds(off, STRIPE)])

    @pl.when(s == NS - 1)
    def _():
        base = (NS - 1) * STRIPE
        pltpu.sync_copy(num_acc.at[pl.ds(base, STRIPE_LAST)],
                        num_out.at[c, pl.ds(base, STRIPE_LAST)])


def _edge_pass(s0, s1, t0, t1, srcs, tgts, znum):
    mesh = plsc.VectorSubcoreMesh(core_axis_name="c", subcore_axis_name="s")
    f = pl.kernel(
        _edge_body,
        out_type=jax.ShapeDtypeStruct((NC, N, OW), jnp.float32),
        mesh=mesh,
        scratch_types=(
            [
                pltpu.VMEM((C,), jnp.int32),
                pltpu.VMEM((C,), jnp.int32),
                pltpu.VMEM((C, W2P), jnp.float32),
                pltpu.VMEM((C, W2P), jnp.float32),
                pltpu.VMEM((C, OW), jnp.float32),
                pltpu.VMEM((C, H + 1), jnp.float32),
                pltpu.VMEM((C,), jnp.int32),
            ] * 2
            + [
                pltpu.VMEM_SHARED((N, OW), jnp.float32),
                pltpu.SemaphoreType.DMA,
                pltpu.SemaphoreType.DMA,
                pltpu.SemaphoreType.DMA,
                pltpu.SemaphoreType.DMA,
                pltpu.SemaphoreType.DMA,
                pltpu.SemaphoreType.DMA,
            ]
        ),
        compiler_params=pltpu.CompilerParams(use_tc_tiling_on_sc=False,
                                             needs_layout_passes=False),
    )
    return f(s0, s1, t0, t1, srcs, tgts, znum)


def _norm_body(acc_ref, out_ref):
    acc = acc_ref[0] + acc_ref[1]
    num = acc[:, :HID]
    den = acc[:, HID:]
    recip = jnp.where(den > 0, 1.0 / den, 0.0)
    row = lax.broadcasted_iota(jnp.int32, (H, HID), 0)
    col = lax.broadcasted_iota(jnp.int32, (H, HID), 1)
    emat = (col // L == row).astype(jnp.float32)
    out_ref[...] = num * jnp.dot(recip, emat,
                                 preferred_element_type=jnp.float32)


def _normalize(num_part):
    return pl.pallas_call(
        _norm_body,
        grid=(N // _R,),
        in_specs=[pl.BlockSpec((NC, _R, OW), lambda i: (0, i, 0))],
        out_specs=pl.BlockSpec((_R, HID), lambda i: (i, 0)),
        out_shape=jax.ShapeDtypeStruct((N, HID), jnp.float32),
    )(num_part)


def kernel(x, adj_lists, Wq, Wk, Wmsg, bmsg):
    srcs = adj_lists[..., 0]
    tgts = adj_lists[..., 1]
    s0, s1, t0, t1 = _build_tables(x, Wq, Wk, Wmsg, bmsg)
    znum = jnp.zeros((N, OW), jnp.float32)
    num_part = _edge_pass(s0, s1, t0, t1, srcs, tgts, znum)
    return _normalize(num_part)
